# baseline (speedup 1.0000x reference)
import functools

import jax
import jax.numpy as jnp
from jax import lax
from jax.experimental import pallas as pl
from jax.experimental.pallas import tpu as pltpu
from jax.experimental.pallas import tpu_sc as plsc

N = 10000
E = 320000
D = 128
NC, NS, L = 2, 16, 16
NW = NC * NS
E_PER_W = E // NW
CHUNK = 128
N_CHUNKS = 80
E_PAD_W = N_CHUNKS * CHUNK
N_PAD = 10008
ROWS_PER_TILE = 624
ROWS_REM = N_PAD - NS * ROWS_PER_TILE


def _edge_sweep_body(weighted, *refs):
    if weighted:
        (rows_hbm, src_hbm, dst_hbm, zr_hbm, zc_hbm, as_hbm, ad_hbm,
         accp_hbm, denp_hbm, *rest) = refs
    else:
        (rows_hbm, src_hbm, dst_hbm, zr_hbm, zc_hbm,
         accp_hbm, denp_hbm, *rest) = refs
        as_hbm = ad_hbm = None
    (isa, ida, isb, idb, rows_a, rows_b, wa, wb,
     asa, ada, asb, adb,
     acc_sh, den_sh,
     sem_ia, sem_ib, sem_ga, sem_gb) = rest

    c = lax.axis_index("c")
    s = lax.axis_index("s")
    wid = s * NC + c

    pltpu.sync_copy(zr_hbm.at[pl.ds(s * ROWS_PER_TILE, ROWS_PER_TILE)],
                    acc_sh.at[pl.ds(s * ROWS_PER_TILE, ROWS_PER_TILE)])

    @pl.when(s == 0)
    def _():
        pltpu.sync_copy(zr_hbm.at[pl.ds(0, ROWS_REM)],
                        acc_sh.at[pl.ds(NS * ROWS_PER_TILE, ROWS_REM)])
        pltpu.sync_copy(zc_hbm, den_sh)

    if not weighted:
        for g in range(CHUNK // L):
            wa[pl.ds(g * L, L)] = jnp.ones((L,), jnp.float32)
            wb[pl.ds(g * L, L)] = jnp.ones((L,), jnp.float32)

    plsc.subcore_barrier()

    base_w = wid * E_PAD_W

    def start_idx(ci, idxs, idxd, sem):
        base = base_w + ci * CHUNK
        pltpu.async_copy(src_hbm.at[pl.ds(base, CHUNK)], idxs, sem)
        pltpu.async_copy(dst_hbm.at[pl.ds(base, CHUNK)], idxd, sem)

    def wait_idx(idxs, idxd, sem):
        pltpu.make_async_copy(src_hbm.at[pl.ds(0, CHUNK)], idxs, sem).wait()
        pltpu.make_async_copy(dst_hbm.at[pl.ds(0, CHUNK)], idxd, sem).wait()

    def start_gath(idxs, idxd, rows_v, asv, adv, sem):
        pltpu.async_copy(rows_hbm.at[idxs], rows_v, sem)
        if weighted:
            pltpu.async_copy(as_hbm.at[idxs], asv, sem)
            pltpu.async_copy(ad_hbm.at[idxd], adv, sem)

    def wait_gath(idxs, idxd, rows_v, asv, adv, sem):
        pltpu.make_async_copy(rows_hbm.at[idxs], rows_v, sem).wait()
        if weighted:
            pltpu.make_async_copy(as_hbm.at[idxs], asv, sem).wait()
            pltpu.make_async_copy(ad_hbm.at[idxd], adv, sem).wait()

    def compute(rows_v, wv, asv, adv):
        if not weighted:
            return

        def scale_group(g, carry2):
            e = asv[pl.ds(g * L, L)] + adv[pl.ds(g * L, L)]
            e = jnp.where(e > 0, e, 0.2 * e)
            wgroup = jnp.exp(e)
            wv[pl.ds(g * L, L)] = wgroup
            for j16 in range(L):
                w = wgroup[j16]
                j = g * L + j16
                for r in range(D // L):
                    rows_v[j, pl.ds(r * L, L)] = rows_v[j, pl.ds(r * L, L)] * w
            return carry2

        lax.fori_loop(0, CHUNK // L, scale_group, 0)

    def scatter(rows_v, wv, idxd):
        pltpu.sync_copy(rows_v, acc_sh.at[idxd], add=True)
        pltpu.sync_copy(wv, den_sh.at[idxd], add=True)

    start_idx(0, isa, ida, sem_ia)
    wait_idx(isa, ida, sem_ia)
    start_gath(isa, ida, rows_a, asa, ada, sem_ga)
    start_idx(1, isb, idb, sem_ib)

    def pair_body(k, carry):
        i0 = 2 * k
        wait_idx(isb, idb, sem_ib)
        start_gath(isb, idb, rows_b, asb, adb, sem_gb)
        wait_gath(isa, ida, rows_a, asa, ada, sem_ga)
        compute(rows_a, wa, asa, ada)
        scatter(rows_a, wa, ida)

        @pl.when(i0 + 2 < N_CHUNKS)
        def _():
            start_idx(i0 + 2, isa, ida, sem_ia)
            wait_idx(isa, ida, sem_ia)
            start_gath(isa, ida, rows_a, asa, ada, sem_ga)

        wait_gath(isb, idb, rows_b, asb, adb, sem_gb)
        compute(rows_b, wb, asb, adb)
        scatter(rows_b, wb, idb)

        @pl.when(i0 + 3 < N_CHUNKS)
        def _():
            start_idx(i0 + 3, isb, idb, sem_ib)

        return carry

    lax.fori_loop(0, N_CHUNKS // 2, pair_body, 0)

    plsc.subcore_barrier()

    pltpu.sync_copy(acc_sh.at[pl.ds(s * ROWS_PER_TILE, ROWS_PER_TILE)],
                    accp_hbm.at[c, pl.ds(s * ROWS_PER_TILE, ROWS_PER_TILE)])

    @pl.when(s == 0)
    def _():
        pltpu.sync_copy(acc_sh.at[pl.ds(NS * ROWS_PER_TILE, ROWS_REM)],
                        accp_hbm.at[c, pl.ds(NS * ROWS_PER_TILE, ROWS_REM)])
        pltpu.sync_copy(den_sh, denp_hbm.at[c])


def _edge_sweep(weighted, rows, src, dst, a_s=None, a_d=None):
    mesh = plsc.VectorSubcoreMesh(core_axis_name="c", subcore_axis_name="s",
                                  num_cores=NC, num_subcores=NS)
    scratch = [
        pltpu.VMEM((CHUNK,), jnp.int32),
        pltpu.VMEM((CHUNK,), jnp.int32),
        pltpu.VMEM((CHUNK,), jnp.int32),
        pltpu.VMEM((CHUNK,), jnp.int32),
        pltpu.VMEM((CHUNK, D), jnp.float32),
        pltpu.VMEM((CHUNK, D), jnp.float32),
        pltpu.VMEM((CHUNK,), jnp.float32),
        pltpu.VMEM((CHUNK,), jnp.float32),
        pltpu.VMEM((CHUNK,), jnp.float32),
        pltpu.VMEM((CHUNK,), jnp.float32),
        pltpu.VMEM((CHUNK,), jnp.float32),
        pltpu.VMEM((CHUNK,), jnp.float32),
        pltpu.VMEM_SHARED((N_PAD, D), jnp.float32),
        pltpu.VMEM_SHARED((N_PAD,), jnp.float32),
        pltpu.SemaphoreType.DMA,
        pltpu.SemaphoreType.DMA,
        pltpu.SemaphoreType.DMA,
        pltpu.SemaphoreType.DMA,
    ]
    fn = pl.kernel(
        functools.partial(_edge_sweep_body, weighted),
        out_type=(
            jax.ShapeDtypeStruct((NC, N_PAD, D), jnp.float32),
            jax.ShapeDtypeStruct((NC, N_PAD), jnp.float32),
        ),
        mesh=mesh,
        scratch_types=scratch,
        compiler_params=pltpu.CompilerParams(needs_layout_passes=False),
    )
    pad_w = E_PAD_W - E_PER_W
    srcw = src.reshape(NW, E_PER_W)
    dstw = dst.reshape(NW, E_PER_W)
    srcf = jnp.pad(srcw, ((0, 0), (0, pad_w))).reshape(NW * E_PAD_W)
    dstf = jnp.pad(dstw, ((0, 0), (0, pad_w)),
                   constant_values=N).reshape(NW * E_PAD_W)
    zr = jnp.zeros((N_PAD, D), jnp.float32)
    zc = jnp.zeros((N_PAD,), jnp.float32)
    if weighted:
        return fn(rows, srcf, dstf, zr, zc,
                  jnp.pad(a_s, (0, N_PAD - N)), jnp.pad(a_d, (0, N_PAD - N)))
    return fn(rows, srcf, dstf, zr, zc)


def _dense_mid_body(accp, cntp, x, wl, wr, gw, bl, ats, atd, hg_o, as_o, ad_o):
    agg = accp[0] + accp[1]
    cnt = cntp[0] + cntp[1]
    mean = agg / jnp.maximum(cnt, 1.0)
    dn = (((1,), (1,)), ((), ()))
    sage = (lax.dot_general(mean, wl[...], dn, preferred_element_type=jnp.float32)
            + bl[...]
            + lax.dot_general(x[...], wr[...], dn, preferred_element_type=jnp.float32))
    h = jnp.maximum(sage, 0.0)
    hg = lax.dot_general(h, gw[...], dn, preferred_element_type=jnp.float32)
    hg_o[...] = hg
    as_o[...] = jnp.sum(hg * ats[...], axis=1, keepdims=True)
    ad_o[...] = jnp.sum(hg * atd[...], axis=1, keepdims=True)


def _dense_final_body(nump, denp, hg, a_s, a_d, bias, out_o):
    num = nump[0] + nump[1]
    den = denp[0] + denp[1]
    es = a_s[...] + a_d[...]
    es = jnp.where(es > 0, es, 0.2 * es)
    ws = jnp.exp(es)
    num = num + ws * hg[...]
    den = den + ws
    o = num / (den + 1e-16) + bias[...]
    m = jnp.max(o, axis=1, keepdims=True)
    lse = jnp.log(jnp.sum(jnp.exp(o - m), axis=1, keepdims=True)) + m
    out_o[...] = o - lse


_BR = 400
_GRID = N // _BR


def _dense_mid(accp, cntp, x, wl, wr, gw, bl, ats, atd):
    return pl.pallas_call(
        _dense_mid_body,
        grid=(_GRID,),
        in_specs=[
            pl.BlockSpec((NC, _BR, D), lambda i: (0, i, 0)),
            pl.BlockSpec((NC, _BR, 1), lambda i: (0, i, 0)),
            pl.BlockSpec((_BR, D), lambda i: (i, 0)),
            pl.BlockSpec((D, D), lambda i: (0, 0)),
            pl.BlockSpec((D, D), lambda i: (0, 0)),
            pl.BlockSpec((D, D), lambda i: (0, 0)),
            pl.BlockSpec((1, D), lambda i: (0, 0)),
            pl.BlockSpec((1, D), lambda i: (0, 0)),
            pl.BlockSpec((1, D), lambda i: (0, 0)),
        ],
        out_specs=[
            pl.BlockSpec((_BR, D), lambda i: (i, 0)),
            pl.BlockSpec((_BR, 1), lambda i: (i, 0)),
            pl.BlockSpec((_BR, 1), lambda i: (i, 0)),
        ],
        out_shape=[
            jax.ShapeDtypeStruct((N, D), jnp.float32),
            jax.ShapeDtypeStruct((N, 1), jnp.float32),
            jax.ShapeDtypeStruct((N, 1), jnp.float32),
        ],
    )(accp, cntp, x, wl, wr, gw, bl, ats, atd)


def _dense_final(nump, denp, hg, a_s, a_d, bias):
    return pl.pallas_call(
        _dense_final_body,
        grid=(_GRID,),
        in_specs=[
            pl.BlockSpec((NC, _BR, D), lambda i: (0, i, 0)),
            pl.BlockSpec((NC, _BR, 1), lambda i: (0, i, 0)),
            pl.BlockSpec((_BR, D), lambda i: (i, 0)),
            pl.BlockSpec((_BR, 1), lambda i: (i, 0)),
            pl.BlockSpec((_BR, 1), lambda i: (i, 0)),
            pl.BlockSpec((1, D), lambda i: (0, 0)),
        ],
        out_specs=pl.BlockSpec((_BR, D), lambda i: (i, 0)),
        out_shape=jax.ShapeDtypeStruct((N, D), jnp.float32),
    )(nump, denp, hg, a_s, a_d, bias)


def kernel(x, edge_index, sage_w_l, sage_b_l, sage_w_r, gat_w, gat_att_src,
           gat_att_dst, gat_bias):
    src = edge_index[0].astype(jnp.int32)
    dst = edge_index[1].astype(jnp.int32)

    accp, cntp = _edge_sweep(False, x, src, dst)

    hg, a_s, a_d = _dense_mid(
        accp, cntp.reshape(NC, N_PAD, 1), x,
        sage_w_l, sage_w_r, gat_w,
        sage_b_l.reshape(1, D),
        gat_att_src.reshape(1, D), gat_att_dst.reshape(1, D))

    nump, denp = _edge_sweep(True, hg, src, dst,
                             a_s.reshape(N), a_d.reshape(N))

    return _dense_final(nump, denp.reshape(NC, N_PAD, 1), hg, a_s, a_d,
                        gat_bias.reshape(1, D))


# speedup vs baseline: 1.7788x; 1.7788x over previous
import functools

import jax
import jax.numpy as jnp
from jax import lax
from jax.experimental import pallas as pl
from jax.experimental.pallas import tpu as pltpu
from jax.experimental.pallas import tpu_sc as plsc

N = 10000
E = 320000
D = 128
NC, NS, L = 2, 16, 16
NW = NC * NS
E_PER_W = E // NW
CHUNK = 80
N_CHUNKS = E_PER_W // CHUNK
N_PAD = 10008
ROWS_PER_TILE = 624
ROWS_REM = N_PAD - NS * ROWS_PER_TILE


def _edge_sweep_body(weighted, *refs):
    if weighted:
        (rows_hbm, src_hbm, dst_hbm, zr_hbm, zc_hbm, as_hbm, ad_hbm,
         accp_hbm, denp_hbm, *rest) = refs
    else:
        (rows_hbm, src_hbm, dst_hbm, zr_hbm, zc_hbm,
         accp_hbm, denp_hbm, *rest) = refs
        as_hbm = ad_hbm = None
    (isa, ida, isb, idb, rows_a, rows_b, wa, wb,
     as_v, ad_v,
     acc_sh, den_sh,
     sem_ga, sem_gb) = rest

    c = lax.axis_index("c")
    s = lax.axis_index("s")
    wid = s * NC + c

    pltpu.sync_copy(zr_hbm.at[pl.ds(s * ROWS_PER_TILE, ROWS_PER_TILE)],
                    acc_sh.at[pl.ds(s * ROWS_PER_TILE, ROWS_PER_TILE)])

    @pl.when(s == 0)
    def _():
        pltpu.sync_copy(zr_hbm.at[pl.ds(0, ROWS_REM)],
                        acc_sh.at[pl.ds(NS * ROWS_PER_TILE, ROWS_REM)])
        pltpu.sync_copy(zc_hbm, den_sh)

    if weighted:
        pltpu.sync_copy(as_hbm, as_v)
        pltpu.sync_copy(ad_hbm, ad_v)
    else:
        for g in range(CHUNK // L):
            wa[pl.ds(g * L, L)] = jnp.ones((L,), jnp.float32)
            wb[pl.ds(g * L, L)] = jnp.ones((L,), jnp.float32)

    plsc.subcore_barrier()

    base_w = wid * E_PER_W

    def load_idx(ci, idxs, idxd):
        base = base_w + ci * CHUNK
        pltpu.sync_copy(src_hbm.at[pl.ds(base, CHUNK)], idxs)
        pltpu.sync_copy(dst_hbm.at[pl.ds(base, CHUNK)], idxd)

    def start_gath(idxs, rows_v, sem):
        pltpu.async_copy(rows_hbm.at[idxs], rows_v, sem)

    def wait_gath(idxs, rows_v, sem):
        pltpu.make_async_copy(rows_hbm.at[idxs], rows_v, sem).wait()

    def compute(rows_v, wv, idxs, idxd):
        if not weighted:
            return

        def scale_group(g, carry2):
            sv = idxs[pl.ds(g * L, L)]
            dv = idxd[pl.ds(g * L, L)]
            e = plsc.load_gather(as_v, [sv]) + plsc.load_gather(ad_v, [dv])
            e = jnp.where(e > 0, e, 0.2 * e)
            wgroup = jnp.exp(e)
            wv[pl.ds(g * L, L)] = wgroup
            for j16 in range(L):
                w = wgroup[j16]
                j = g * L + j16
                for r in range(D // L):
                    rows_v[j, pl.ds(r * L, L)] = rows_v[j, pl.ds(r * L, L)] * w
            return carry2

        lax.fori_loop(0, CHUNK // L, scale_group, 0)

    def scatter(rows_v, wv, idxd):
        pltpu.sync_copy(rows_v, acc_sh.at[idxd], add=True)
        pltpu.sync_copy(wv, den_sh.at[idxd], add=True)

    load_idx(0, isa, ida)
    start_gath(isa, rows_a, sem_ga)
    load_idx(1, isb, idb)
    start_gath(isb, rows_b, sem_gb)

    def pair_body(k, carry):
        i0 = 2 * k
        wait_gath(isa, rows_a, sem_ga)
        compute(rows_a, wa, isa, ida)
        scatter(rows_a, wa, ida)

        @pl.when(i0 + 2 < N_CHUNKS)
        def _():
            load_idx(i0 + 2, isa, ida)
            start_gath(isa, rows_a, sem_ga)

        wait_gath(isb, rows_b, sem_gb)
        compute(rows_b, wb, isb, idb)
        scatter(rows_b, wb, idb)

        @pl.when(i0 + 3 < N_CHUNKS)
        def _():
            load_idx(i0 + 3, isb, idb)
            start_gath(isb, rows_b, sem_gb)

        return carry

    lax.fori_loop(0, N_CHUNKS // 2, pair_body, 0)
    if N_CHUNKS % 2:
        wait_gath(isa, rows_a, sem_ga)
        compute(rows_a, wa, isa, ida)
        scatter(rows_a, wa, ida)

    plsc.subcore_barrier()

    pltpu.sync_copy(acc_sh.at[pl.ds(s * ROWS_PER_TILE, ROWS_PER_TILE)],
                    accp_hbm.at[c, pl.ds(s * ROWS_PER_TILE, ROWS_PER_TILE)])

    @pl.when(s == 0)
    def _():
        pltpu.sync_copy(acc_sh.at[pl.ds(NS * ROWS_PER_TILE, ROWS_REM)],
                        accp_hbm.at[c, pl.ds(NS * ROWS_PER_TILE, ROWS_REM)])
        pltpu.sync_copy(den_sh, denp_hbm.at[c])


def _edge_sweep(weighted, rows, src, dst, a_s=None, a_d=None):
    mesh = plsc.VectorSubcoreMesh(core_axis_name="c", subcore_axis_name="s",
                                  num_cores=NC, num_subcores=NS)
    scratch = [
        pltpu.VMEM((CHUNK,), jnp.int32),
        pltpu.VMEM((CHUNK,), jnp.int32),
        pltpu.VMEM((CHUNK,), jnp.int32),
        pltpu.VMEM((CHUNK,), jnp.int32),
        pltpu.VMEM((CHUNK, D), jnp.float32),
        pltpu.VMEM((CHUNK, D), jnp.float32),
        pltpu.VMEM((CHUNK,), jnp.float32),
        pltpu.VMEM((CHUNK,), jnp.float32),
        pltpu.VMEM((N_PAD if weighted else 8,), jnp.float32),
        pltpu.VMEM((N_PAD if weighted else 8,), jnp.float32),
        pltpu.VMEM_SHARED((N_PAD, D), jnp.float32),
        pltpu.VMEM_SHARED((N_PAD,), jnp.float32),
        pltpu.SemaphoreType.DMA,
        pltpu.SemaphoreType.DMA,
    ]
    fn = pl.kernel(
        functools.partial(_edge_sweep_body, weighted),
        out_type=(
            jax.ShapeDtypeStruct((NC, N_PAD, D), jnp.float32),
            jax.ShapeDtypeStruct((NC, N_PAD), jnp.float32),
        ),
        mesh=mesh,
        scratch_types=scratch,
        compiler_params=pltpu.CompilerParams(needs_layout_passes=False),
    )
    srcf = src
    dstf = dst
    zr = jnp.zeros((N_PAD, D), jnp.float32)
    zc = jnp.zeros((N_PAD,), jnp.float32)
    if weighted:
        return fn(rows, srcf, dstf, zr, zc,
                  jnp.pad(a_s, (0, N_PAD - N)), jnp.pad(a_d, (0, N_PAD - N)))
    return fn(rows, srcf, dstf, zr, zc)


def _dense_mid_body(accp, cntp, x, wl, wr, gw, bl, ats, atd, hg_o, as_o, ad_o):
    agg = accp[0] + accp[1]
    cnt = cntp[0] + cntp[1]
    mean = agg / jnp.maximum(cnt, 1.0)
    dn = (((1,), (1,)), ((), ()))
    sage = (lax.dot_general(mean, wl[...], dn, preferred_element_type=jnp.float32)
            + bl[...]
            + lax.dot_general(x[...], wr[...], dn, preferred_element_type=jnp.float32))
    h = jnp.maximum(sage, 0.0)
    hg = lax.dot_general(h, gw[...], dn, preferred_element_type=jnp.float32)
    hg_o[...] = hg
    as_o[...] = jnp.sum(hg * ats[...], axis=1, keepdims=True)
    ad_o[...] = jnp.sum(hg * atd[...], axis=1, keepdims=True)


def _dense_final_body(nump, denp, hg, a_s, a_d, bias, out_o):
    num = nump[0] + nump[1]
    den = denp[0] + denp[1]
    es = a_s[...] + a_d[...]
    es = jnp.where(es > 0, es, 0.2 * es)
    ws = jnp.exp(es)
    num = num + ws * hg[...]
    den = den + ws
    o = num / (den + 1e-16) + bias[...]
    m = jnp.max(o, axis=1, keepdims=True)
    lse = jnp.log(jnp.sum(jnp.exp(o - m), axis=1, keepdims=True)) + m
    out_o[...] = o - lse


_BR = 400
_GRID = N // _BR


def _dense_mid(accp, cntp, x, wl, wr, gw, bl, ats, atd):
    return pl.pallas_call(
        _dense_mid_body,
        grid=(_GRID,),
        in_specs=[
            pl.BlockSpec((NC, _BR, D), lambda i: (0, i, 0)),
            pl.BlockSpec((NC, _BR, 1), lambda i: (0, i, 0)),
            pl.BlockSpec((_BR, D), lambda i: (i, 0)),
            pl.BlockSpec((D, D), lambda i: (0, 0)),
            pl.BlockSpec((D, D), lambda i: (0, 0)),
            pl.BlockSpec((D, D), lambda i: (0, 0)),
            pl.BlockSpec((1, D), lambda i: (0, 0)),
            pl.BlockSpec((1, D), lambda i: (0, 0)),
            pl.BlockSpec((1, D), lambda i: (0, 0)),
        ],
        out_specs=[
            pl.BlockSpec((_BR, D), lambda i: (i, 0)),
            pl.BlockSpec((_BR, 1), lambda i: (i, 0)),
            pl.BlockSpec((_BR, 1), lambda i: (i, 0)),
        ],
        out_shape=[
            jax.ShapeDtypeStruct((N, D), jnp.float32),
            jax.ShapeDtypeStruct((N, 1), jnp.float32),
            jax.ShapeDtypeStruct((N, 1), jnp.float32),
        ],
    )(accp, cntp, x, wl, wr, gw, bl, ats, atd)


def _dense_final(nump, denp, hg, a_s, a_d, bias):
    return pl.pallas_call(
        _dense_final_body,
        grid=(_GRID,),
        in_specs=[
            pl.BlockSpec((NC, _BR, D), lambda i: (0, i, 0)),
            pl.BlockSpec((NC, _BR, 1), lambda i: (0, i, 0)),
            pl.BlockSpec((_BR, D), lambda i: (i, 0)),
            pl.BlockSpec((_BR, 1), lambda i: (i, 0)),
            pl.BlockSpec((_BR, 1), lambda i: (i, 0)),
            pl.BlockSpec((1, D), lambda i: (0, 0)),
        ],
        out_specs=pl.BlockSpec((_BR, D), lambda i: (i, 0)),
        out_shape=jax.ShapeDtypeStruct((N, D), jnp.float32),
    )(nump, denp, hg, a_s, a_d, bias)


def kernel(x, edge_index, sage_w_l, sage_b_l, sage_w_r, gat_w, gat_att_src,
           gat_att_dst, gat_bias):
    src = edge_index[0].astype(jnp.int32)
    dst = edge_index[1].astype(jnp.int32)

    accp, cntp = _edge_sweep(False, x, src, dst)

    hg, a_s, a_d = _dense_mid(
        accp, cntp.reshape(NC, N_PAD, 1), x,
        sage_w_l, sage_w_r, gat_w,
        sage_b_l.reshape(1, D),
        gat_att_src.reshape(1, D), gat_att_dst.reshape(1, D))

    nump, denp = _edge_sweep(True, hg, src, dst,
                             a_s.reshape(N), a_d.reshape(N))

    return _dense_final(nump, denp.reshape(NC, N_PAD, 1), hg, a_s, a_d,
                        gat_bias.reshape(1, D))


# speedup vs baseline: 1.8438x; 1.0365x over previous
import functools

import jax
import jax.numpy as jnp
from jax import lax
from jax.experimental import pallas as pl
from jax.experimental.pallas import tpu as pltpu
from jax.experimental.pallas import tpu_sc as plsc

N = 10000
E = 320000
D = 128
NC, NS, L = 2, 16, 16
NW = NC * NS
E_PER_W = E // NW
CHUNK = 80
N_CHUNKS = E_PER_W // CHUNK
N_PAD = 10008
ROWS_PER_TILE = 624
ROWS_REM = N_PAD - NS * ROWS_PER_TILE


def _edge_sweep_body(weighted, *refs):
    if weighted:
        (rows_hbm, src_hbm, dst_hbm, zr_hbm, zc_hbm, as_hbm, ad_hbm,
         accp_hbm, denp_hbm, *rest) = refs
    else:
        (rows_hbm, src_hbm, dst_hbm, zr_hbm, zc_hbm,
         accp_hbm, denp_hbm, *rest) = refs
        as_hbm = ad_hbm = None
    (isa, ida, isb, idb, rows_a, rows_b, wa, wb,
     as_v, ad_v,
     acc_sh, den_sh,
     sem_ga, sem_gb, sem_sa, sem_sb) = rest

    c = lax.axis_index("c")
    s = lax.axis_index("s")
    wid = s * NC + c

    pltpu.sync_copy(zr_hbm.at[pl.ds(s * ROWS_PER_TILE, ROWS_PER_TILE)],
                    acc_sh.at[pl.ds(s * ROWS_PER_TILE, ROWS_PER_TILE)])

    @pl.when(s == 0)
    def _():
        pltpu.sync_copy(zr_hbm.at[pl.ds(0, ROWS_REM)],
                        acc_sh.at[pl.ds(NS * ROWS_PER_TILE, ROWS_REM)])
        pltpu.sync_copy(zc_hbm, den_sh)

    if weighted:
        pltpu.sync_copy(as_hbm, as_v)
        pltpu.sync_copy(ad_hbm, ad_v)
    else:
        for g in range(CHUNK // L):
            wa[pl.ds(g * L, L)] = jnp.ones((L,), jnp.float32)
            wb[pl.ds(g * L, L)] = jnp.ones((L,), jnp.float32)

    plsc.subcore_barrier()

    base_w = wid * E_PER_W

    def load_idx(ci, idxs, idxd):
        base = base_w + ci * CHUNK
        pltpu.sync_copy(src_hbm.at[pl.ds(base, CHUNK)], idxs)
        pltpu.sync_copy(dst_hbm.at[pl.ds(base, CHUNK)], idxd)

    def start_gath(idxs, rows_v, sem):
        pltpu.async_copy(rows_hbm.at[idxs], rows_v, sem)

    def wait_gath(idxs, rows_v, sem):
        pltpu.make_async_copy(rows_hbm.at[idxs], rows_v, sem).wait()

    def compute(rows_v, wv, idxs, idxd):
        if not weighted:
            return

        def scale_group(g, carry2):
            sv = idxs[pl.ds(g * L, L)]
            dv = idxd[pl.ds(g * L, L)]
            e = plsc.load_gather(as_v, [sv]) + plsc.load_gather(ad_v, [dv])
            e = jnp.where(e > 0, e, 0.2 * e)
            wgroup = jnp.exp(e)
            wv[pl.ds(g * L, L)] = wgroup
            for j16 in range(L):
                w = wgroup[j16]
                j = g * L + j16
                for r in range(D // L):
                    rows_v[j, pl.ds(r * L, L)] = rows_v[j, pl.ds(r * L, L)] * w
            return carry2

        lax.fori_loop(0, CHUNK // L, scale_group, 0)

    def start_scat(rows_v, wv, idxd, sem):
        pltpu.async_copy(rows_v, acc_sh.at[idxd], sem, add=True)
        pltpu.async_copy(wv, den_sh.at[idxd], sem, add=True)

    def wait_scat(rows_v, wv, idxd, sem):
        pltpu.make_async_copy(rows_v, acc_sh.at[idxd], sem).wait()
        pltpu.make_async_copy(wv, den_sh.at[idxd], sem).wait()

    load_idx(0, isa, ida)
    start_gath(isa, rows_a, sem_ga)
    load_idx(1, isb, idb)
    start_gath(isb, rows_b, sem_gb)

    def pair_body(k, carry):
        i0 = 2 * k
        wait_gath(isa, rows_a, sem_ga)
        compute(rows_a, wa, isa, ida)
        start_scat(rows_a, wa, ida, sem_sa)

        wait_gath(isb, rows_b, sem_gb)
        compute(rows_b, wb, isb, idb)
        start_scat(rows_b, wb, idb, sem_sb)

        @pl.when(i0 + 2 < N_CHUNKS)
        def _():
            wait_scat(rows_a, wa, ida, sem_sa)
            load_idx(i0 + 2, isa, ida)
            start_gath(isa, rows_a, sem_ga)

        @pl.when(i0 + 3 < N_CHUNKS)
        def _():
            wait_scat(rows_b, wb, idb, sem_sb)
            load_idx(i0 + 3, isb, idb)
            start_gath(isb, rows_b, sem_gb)

        return carry

    lax.fori_loop(0, N_CHUNKS // 2, pair_body, 0)
    if N_CHUNKS % 2:
        wait_gath(isa, rows_a, sem_ga)
        compute(rows_a, wa, isa, ida)
        start_scat(rows_a, wa, ida, sem_sa)
        wait_scat(rows_a, wa, ida, sem_sa)
    wait_scat(rows_b, wb, idb, sem_sb)
    if N_CHUNKS % 2 == 0:
        wait_scat(rows_a, wa, ida, sem_sa)

    plsc.subcore_barrier()

    pltpu.sync_copy(acc_sh.at[pl.ds(s * ROWS_PER_TILE, ROWS_PER_TILE)],
                    accp_hbm.at[c, pl.ds(s * ROWS_PER_TILE, ROWS_PER_TILE)])

    @pl.when(s == 0)
    def _():
        pltpu.sync_copy(acc_sh.at[pl.ds(NS * ROWS_PER_TILE, ROWS_REM)],
                        accp_hbm.at[c, pl.ds(NS * ROWS_PER_TILE, ROWS_REM)])
        pltpu.sync_copy(den_sh, denp_hbm.at[c])


def _edge_sweep(weighted, rows, src, dst, a_s=None, a_d=None):
    mesh = plsc.VectorSubcoreMesh(core_axis_name="c", subcore_axis_name="s",
                                  num_cores=NC, num_subcores=NS)
    scratch = [
        pltpu.VMEM((CHUNK,), jnp.int32),
        pltpu.VMEM((CHUNK,), jnp.int32),
        pltpu.VMEM((CHUNK,), jnp.int32),
        pltpu.VMEM((CHUNK,), jnp.int32),
        pltpu.VMEM((CHUNK, D), jnp.float32),
        pltpu.VMEM((CHUNK, D), jnp.float32),
        pltpu.VMEM((CHUNK,), jnp.float32),
        pltpu.VMEM((CHUNK,), jnp.float32),
        pltpu.VMEM((N_PAD if weighted else 8,), jnp.float32),
        pltpu.VMEM((N_PAD if weighted else 8,), jnp.float32),
        pltpu.VMEM_SHARED((N_PAD, D), jnp.float32),
        pltpu.VMEM_SHARED((N_PAD,), jnp.float32),
        pltpu.SemaphoreType.DMA,
        pltpu.SemaphoreType.DMA,
        pltpu.SemaphoreType.DMA,
        pltpu.SemaphoreType.DMA,
    ]
    fn = pl.kernel(
        functools.partial(_edge_sweep_body, weighted),
        out_type=(
            jax.ShapeDtypeStruct((NC, N_PAD, D), jnp.float32),
            jax.ShapeDtypeStruct((NC, N_PAD), jnp.float32),
        ),
        mesh=mesh,
        scratch_types=scratch,
        compiler_params=pltpu.CompilerParams(needs_layout_passes=False),
    )
    srcf = src
    dstf = dst
    zr = jnp.zeros((N_PAD, D), jnp.float32)
    zc = jnp.zeros((N_PAD,), jnp.float32)
    if weighted:
        return fn(rows, srcf, dstf, zr, zc,
                  jnp.pad(a_s, (0, N_PAD - N)), jnp.pad(a_d, (0, N_PAD - N)))
    return fn(rows, srcf, dstf, zr, zc)


def _dense_mid_body(accp, cntp, x, wl, wr, gw, bl, ats, atd, hg_o, as_o, ad_o):
    agg = accp[0] + accp[1]
    cnt = cntp[0] + cntp[1]
    mean = agg / jnp.maximum(cnt, 1.0)
    dn = (((1,), (1,)), ((), ()))
    sage = (lax.dot_general(mean, wl[...], dn, preferred_element_type=jnp.float32)
            + bl[...]
            + lax.dot_general(x[...], wr[...], dn, preferred_element_type=jnp.float32))
    h = jnp.maximum(sage, 0.0)
    hg = lax.dot_general(h, gw[...], dn, preferred_element_type=jnp.float32)
    hg_o[...] = hg
    as_o[...] = jnp.sum(hg * ats[...], axis=1, keepdims=True)
    ad_o[...] = jnp.sum(hg * atd[...], axis=1, keepdims=True)


def _dense_final_body(nump, denp, hg, a_s, a_d, bias, out_o):
    num = nump[0] + nump[1]
    den = denp[0] + denp[1]
    es = a_s[...] + a_d[...]
    es = jnp.where(es > 0, es, 0.2 * es)
    ws = jnp.exp(es)
    num = num + ws * hg[...]
    den = den + ws
    o = num / (den + 1e-16) + bias[...]
    m = jnp.max(o, axis=1, keepdims=True)
    lse = jnp.log(jnp.sum(jnp.exp(o - m), axis=1, keepdims=True)) + m
    out_o[...] = o - lse


_BR = 400
_GRID = N // _BR


def _dense_mid(accp, cntp, x, wl, wr, gw, bl, ats, atd):
    return pl.pallas_call(
        _dense_mid_body,
        grid=(_GRID,),
        in_specs=[
            pl.BlockSpec((NC, _BR, D), lambda i: (0, i, 0)),
            pl.BlockSpec((NC, _BR, 1), lambda i: (0, i, 0)),
            pl.BlockSpec((_BR, D), lambda i: (i, 0)),
            pl.BlockSpec((D, D), lambda i: (0, 0)),
            pl.BlockSpec((D, D), lambda i: (0, 0)),
            pl.BlockSpec((D, D), lambda i: (0, 0)),
            pl.BlockSpec((1, D), lambda i: (0, 0)),
            pl.BlockSpec((1, D), lambda i: (0, 0)),
            pl.BlockSpec((1, D), lambda i: (0, 0)),
        ],
        out_specs=[
            pl.BlockSpec((_BR, D), lambda i: (i, 0)),
            pl.BlockSpec((_BR, 1), lambda i: (i, 0)),
            pl.BlockSpec((_BR, 1), lambda i: (i, 0)),
        ],
        out_shape=[
            jax.ShapeDtypeStruct((N, D), jnp.float32),
            jax.ShapeDtypeStruct((N, 1), jnp.float32),
            jax.ShapeDtypeStruct((N, 1), jnp.float32),
        ],
    )(accp, cntp, x, wl, wr, gw, bl, ats, atd)


def _dense_final(nump, denp, hg, a_s, a_d, bias):
    return pl.pallas_call(
        _dense_final_body,
        grid=(_GRID,),
        in_specs=[
            pl.BlockSpec((NC, _BR, D), lambda i: (0, i, 0)),
            pl.BlockSpec((NC, _BR, 1), lambda i: (0, i, 0)),
            pl.BlockSpec((_BR, D), lambda i: (i, 0)),
            pl.BlockSpec((_BR, 1), lambda i: (i, 0)),
            pl.BlockSpec((_BR, 1), lambda i: (i, 0)),
            pl.BlockSpec((1, D), lambda i: (0, 0)),
        ],
        out_specs=pl.BlockSpec((_BR, D), lambda i: (i, 0)),
        out_shape=jax.ShapeDtypeStruct((N, D), jnp.float32),
    )(nump, denp, hg, a_s, a_d, bias)


def kernel(x, edge_index, sage_w_l, sage_b_l, sage_w_r, gat_w, gat_att_src,
           gat_att_dst, gat_bias):
    src = edge_index[0].astype(jnp.int32)
    dst = edge_index[1].astype(jnp.int32)

    accp, cntp = _edge_sweep(False, x, src, dst)

    hg, a_s, a_d = _dense_mid(
        accp, cntp.reshape(NC, N_PAD, 1), x,
        sage_w_l, sage_w_r, gat_w,
        sage_b_l.reshape(1, D),
        gat_att_src.reshape(1, D), gat_att_dst.reshape(1, D))

    nump, denp = _edge_sweep(True, hg, src, dst,
                             a_s.reshape(N), a_d.reshape(N))

    return _dense_final(nump, denp.reshape(NC, N_PAD, 1), hg, a_s, a_d,
                        gat_bias.reshape(1, D))


# speedup vs baseline: 2.1251x; 1.1526x over previous
import functools

import jax
import jax.numpy as jnp
from jax import lax
from jax.experimental import pallas as pl
from jax.experimental.pallas import tpu as pltpu
from jax.experimental.pallas import tpu_sc as plsc

N = 10000
E = 320000
D = 128
NC, NS, L = 2, 16, 16
NW = NC * NS
E_PER_W = E // NW
CHUNK = 80
N_CHUNKS = E_PER_W // CHUNK
N_PAD = 10008
ROWS_PER_TILE = 624
ROWS_REM = N_PAD - NS * ROWS_PER_TILE


def _edge_sweep_body(weighted, *refs):
    if weighted:
        (rows_hbm, ed_hbm, zr_hbm, zc_hbm, as_hbm, ad_hbm,
         accp_hbm, denp_hbm, *rest) = refs
    else:
        (rows_hbm, ed_hbm, zr_hbm, zc_hbm,
         accp_hbm, denp_hbm, *rest) = refs
        as_hbm = ad_hbm = None
    (iba, ida, ibb, idb, rows_a, rows_b, wa, wb,
     as_v, ad_v,
     acc_sh, den_sh,
     sem_ga, sem_gb, sem_sa, sem_sb) = rest

    c = lax.axis_index("c")
    s = lax.axis_index("s")
    wid = s * NC + c

    pltpu.sync_copy(zr_hbm.at[pl.ds(s * ROWS_PER_TILE, ROWS_PER_TILE)],
                    acc_sh.at[pl.ds(s * ROWS_PER_TILE, ROWS_PER_TILE)])

    @pl.when(s == 0)
    def _():
        pltpu.sync_copy(zr_hbm.at[pl.ds(0, ROWS_REM)],
                        acc_sh.at[pl.ds(NS * ROWS_PER_TILE, ROWS_REM)])
        pltpu.sync_copy(zc_hbm, den_sh)

    if weighted:
        pltpu.sync_copy(as_hbm, as_v)
        pltpu.sync_copy(ad_hbm, ad_v)
    else:
        for g in range(CHUNK // L):
            wa[pl.ds(g * L, L)] = jnp.ones((L,), jnp.float32)
            wb[pl.ds(g * L, L)] = jnp.ones((L,), jnp.float32)

    plsc.subcore_barrier()

    base_w = wid * N_CHUNKS

    def load_idx(ci, ibuf, idxd):
        base = (base_w + ci) * 2 * CHUNK
        pltpu.sync_copy(ed_hbm.at[pl.ds(base, 2 * CHUNK)], ibuf)
        for g in range(CHUNK // L):
            idxd[pl.ds(g * L, L)] = ibuf[pl.ds(CHUNK + g * L, L)]

    def start_gath(ibuf, rows_v, sem):
        pltpu.async_copy(rows_hbm.at[ibuf.at[pl.ds(0, CHUNK)]], rows_v, sem)

    def wait_gath(ibuf, rows_v, sem):
        pltpu.make_async_copy(rows_hbm.at[ibuf.at[pl.ds(0, CHUNK)]],
                              rows_v, sem).wait()

    def compute(rows_v, wv, ibuf, idxd):
        if not weighted:
            return

        def scale_group(g, carry2):
            sv = ibuf[pl.ds(g * L, L)]
            dv = idxd[pl.ds(g * L, L)]
            e = plsc.load_gather(as_v, [sv]) + plsc.load_gather(ad_v, [dv])
            e = jnp.where(e > 0, e, 0.2 * e)
            wgroup = jnp.exp(e)
            wv[pl.ds(g * L, L)] = wgroup
            for j16 in range(L):
                w = wgroup[j16]
                j = g * L + j16
                for r in range(D // L):
                    rows_v[j, pl.ds(r * L, L)] = rows_v[j, pl.ds(r * L, L)] * w
            return carry2

        lax.fori_loop(0, CHUNK // L, scale_group, 0)

    def start_scat(rows_v, wv, idxd, sem):
        pltpu.async_copy(rows_v, acc_sh.at[idxd], sem, add=True)
        pltpu.async_copy(wv, den_sh.at[idxd], sem, add=True)

    def wait_scat(rows_v, wv, idxd, sem):
        pltpu.make_async_copy(rows_v, acc_sh.at[idxd], sem).wait()
        pltpu.make_async_copy(wv, den_sh.at[idxd], sem).wait()

    load_idx(0, iba, ida)
    start_gath(iba, rows_a, sem_ga)
    load_idx(1, ibb, idb)
    start_gath(ibb, rows_b, sem_gb)

    def pair_body(k, carry):
        i0 = 2 * k
        wait_gath(iba, rows_a, sem_ga)
        compute(rows_a, wa, iba, ida)
        start_scat(rows_a, wa, ida, sem_sa)

        wait_gath(ibb, rows_b, sem_gb)
        compute(rows_b, wb, ibb, idb)
        start_scat(rows_b, wb, idb, sem_sb)

        @pl.when(i0 + 2 < N_CHUNKS)
        def _():
            wait_scat(rows_a, wa, ida, sem_sa)
            load_idx(i0 + 2, iba, ida)
            start_gath(iba, rows_a, sem_ga)

        @pl.when(i0 + 3 < N_CHUNKS)
        def _():
            wait_scat(rows_b, wb, idb, sem_sb)
            load_idx(i0 + 3, ibb, idb)
            start_gath(ibb, rows_b, sem_gb)

        return carry

    lax.fori_loop(0, N_CHUNKS // 2, pair_body, 0)
    if N_CHUNKS % 2:
        wait_gath(iba, rows_a, sem_ga)
        compute(rows_a, wa, iba, ida)
        start_scat(rows_a, wa, ida, sem_sa)
        wait_scat(rows_a, wa, ida, sem_sa)
    wait_scat(rows_b, wb, idb, sem_sb)
    if N_CHUNKS % 2 == 0:
        wait_scat(rows_a, wa, ida, sem_sa)

    plsc.subcore_barrier()

    pltpu.sync_copy(acc_sh.at[pl.ds(s * ROWS_PER_TILE, ROWS_PER_TILE)],
                    accp_hbm.at[c, pl.ds(s * ROWS_PER_TILE, ROWS_PER_TILE)])

    @pl.when(s == 0)
    def _():
        pltpu.sync_copy(acc_sh.at[pl.ds(NS * ROWS_PER_TILE, ROWS_REM)],
                        accp_hbm.at[c, pl.ds(NS * ROWS_PER_TILE, ROWS_REM)])
        pltpu.sync_copy(den_sh, denp_hbm.at[c])


def _edge_sweep(weighted, rows, src, dst, a_s=None, a_d=None):
    mesh = plsc.VectorSubcoreMesh(core_axis_name="c", subcore_axis_name="s",
                                  num_cores=NC, num_subcores=NS)
    scratch = [
        pltpu.VMEM((2 * CHUNK,), jnp.int32),
        pltpu.VMEM((CHUNK,), jnp.int32),
        pltpu.VMEM((2 * CHUNK,), jnp.int32),
        pltpu.VMEM((CHUNK,), jnp.int32),
        pltpu.VMEM((CHUNK, D), jnp.float32),
        pltpu.VMEM((CHUNK, D), jnp.float32),
        pltpu.VMEM((CHUNK,), jnp.float32),
        pltpu.VMEM((CHUNK,), jnp.float32),
        pltpu.VMEM((N_PAD if weighted else 8,), jnp.float32),
        pltpu.VMEM((N_PAD if weighted else 8,), jnp.float32),
        pltpu.VMEM_SHARED((N_PAD, D), jnp.float32),
        pltpu.VMEM_SHARED((N_PAD,), jnp.float32),
        pltpu.SemaphoreType.DMA,
        pltpu.SemaphoreType.DMA,
        pltpu.SemaphoreType.DMA,
        pltpu.SemaphoreType.DMA,
    ]
    fn = pl.kernel(
        functools.partial(_edge_sweep_body, weighted),
        out_type=(
            jax.ShapeDtypeStruct((NC, N_PAD, D), jnp.float32),
            jax.ShapeDtypeStruct((NC, N_PAD), jnp.float32),
        ),
        mesh=mesh,
        scratch_types=scratch,
        compiler_params=pltpu.CompilerParams(needs_layout_passes=False),
    )
    ed = jnp.stack([src.reshape(NW, N_CHUNKS, CHUNK),
                    dst.reshape(NW, N_CHUNKS, CHUNK)], axis=2).reshape(-1)
    zr = jnp.zeros((N_PAD, D), jnp.float32)
    zc = jnp.zeros((N_PAD,), jnp.float32)
    if weighted:
        return fn(rows, ed, zr, zc,
                  jnp.pad(a_s, (0, N_PAD - N)), jnp.pad(a_d, (0, N_PAD - N)))
    return fn(rows, ed, zr, zc)


def _dense_mid_body(accp, cntp, x, wl, wr, gw, bl, ats, atd, hg_o, as_o, ad_o):
    agg = accp[0] + accp[1]
    cnt = cntp[0] + cntp[1]
    mean = agg / jnp.maximum(cnt, 1.0)
    dn = (((1,), (1,)), ((), ()))
    sage = (lax.dot_general(mean, wl[...], dn, preferred_element_type=jnp.float32)
            + bl[...]
            + lax.dot_general(x[...], wr[...], dn, preferred_element_type=jnp.float32))
    h = jnp.maximum(sage, 0.0)
    hg = lax.dot_general(h, gw[...], dn, preferred_element_type=jnp.float32)
    hg_o[...] = hg
    as_o[...] = jnp.sum(hg * ats[...], axis=1, keepdims=True)
    ad_o[...] = jnp.sum(hg * atd[...], axis=1, keepdims=True)


def _dense_final_body(nump, denp, hg, a_s, a_d, bias, out_o):
    num = nump[0] + nump[1]
    den = denp[0] + denp[1]
    es = a_s[...] + a_d[...]
    es = jnp.where(es > 0, es, 0.2 * es)
    ws = jnp.exp(es)
    num = num + ws * hg[...]
    den = den + ws
    o = num / (den + 1e-16) + bias[...]
    m = jnp.max(o, axis=1, keepdims=True)
    lse = jnp.log(jnp.sum(jnp.exp(o - m), axis=1, keepdims=True)) + m
    out_o[...] = o - lse


_BR = 400
_GRID = N // _BR


def _dense_mid(accp, cntp, x, wl, wr, gw, bl, ats, atd):
    return pl.pallas_call(
        _dense_mid_body,
        grid=(_GRID,),
        in_specs=[
            pl.BlockSpec((NC, _BR, D), lambda i: (0, i, 0)),
            pl.BlockSpec((NC, _BR, 1), lambda i: (0, i, 0)),
            pl.BlockSpec((_BR, D), lambda i: (i, 0)),
            pl.BlockSpec((D, D), lambda i: (0, 0)),
            pl.BlockSpec((D, D), lambda i: (0, 0)),
            pl.BlockSpec((D, D), lambda i: (0, 0)),
            pl.BlockSpec((1, D), lambda i: (0, 0)),
            pl.BlockSpec((1, D), lambda i: (0, 0)),
            pl.BlockSpec((1, D), lambda i: (0, 0)),
        ],
        out_specs=[
            pl.BlockSpec((_BR, D), lambda i: (i, 0)),
            pl.BlockSpec((_BR, 1), lambda i: (i, 0)),
            pl.BlockSpec((_BR, 1), lambda i: (i, 0)),
        ],
        out_shape=[
            jax.ShapeDtypeStruct((N, D), jnp.float32),
            jax.ShapeDtypeStruct((N, 1), jnp.float32),
            jax.ShapeDtypeStruct((N, 1), jnp.float32),
        ],
    )(accp, cntp, x, wl, wr, gw, bl, ats, atd)


def _dense_final(nump, denp, hg, a_s, a_d, bias):
    return pl.pallas_call(
        _dense_final_body,
        grid=(_GRID,),
        in_specs=[
            pl.BlockSpec((NC, _BR, D), lambda i: (0, i, 0)),
            pl.BlockSpec((NC, _BR, 1), lambda i: (0, i, 0)),
            pl.BlockSpec((_BR, D), lambda i: (i, 0)),
            pl.BlockSpec((_BR, 1), lambda i: (i, 0)),
            pl.BlockSpec((_BR, 1), lambda i: (i, 0)),
            pl.BlockSpec((1, D), lambda i: (0, 0)),
        ],
        out_specs=pl.BlockSpec((_BR, D), lambda i: (i, 0)),
        out_shape=jax.ShapeDtypeStruct((N, D), jnp.float32),
    )(nump, denp, hg, a_s, a_d, bias)


def kernel(x, edge_index, sage_w_l, sage_b_l, sage_w_r, gat_w, gat_att_src,
           gat_att_dst, gat_bias):
    src = edge_index[0].astype(jnp.int32)
    dst = edge_index[1].astype(jnp.int32)

    accp, cntp = _edge_sweep(False, x, src, dst)

    hg, a_s, a_d = _dense_mid(
        accp, cntp.reshape(NC, N_PAD, 1), x,
        sage_w_l, sage_w_r, gat_w,
        sage_b_l.reshape(1, D),
        gat_att_src.reshape(1, D), gat_att_dst.reshape(1, D))

    nump, denp = _edge_sweep(True, hg, src, dst,
                             a_s.reshape(N), a_d.reshape(N))

    return _dense_final(nump, denp.reshape(NC, N_PAD, 1), hg, a_s, a_d,
                        gat_bias.reshape(1, D))


# speedup vs baseline: 2.2223x; 1.0458x over previous
import functools

import jax
import jax.numpy as jnp
from jax import lax
from jax.experimental import pallas as pl
from jax.experimental.pallas import tpu as pltpu
from jax.experimental.pallas import tpu_sc as plsc

N = 10000
E = 320000
D = 128
NC, NS, L = 2, 16, 16
NW = NC * NS
E_PER_W = E // NW
CHUNK = 80
N_CHUNKS = E_PER_W // CHUNK
N_PAD = 10008
ROWS_PER_TILE = 624
ROWS_REM = N_PAD - NS * ROWS_PER_TILE


def _edge_sweep_body(weighted, *refs):
    if weighted:
        (rows_hbm, ed_hbm, zr_hbm, zc_hbm, as_hbm, ad_hbm,
         accp_hbm, denp_hbm, *rest) = refs
    else:
        (rows_hbm, ed_hbm, zr_hbm, zc_hbm,
         accp_hbm, denp_hbm, *rest) = refs
        as_hbm = ad_hbm = None
    (iba, ida, ibb, idb, rows_a, rows_b, wa, wb,
     as_v, ad_v,
     acc_sh, den_sh,
     sem_ga, sem_gb, sem_sa, sem_sb) = rest

    c = lax.axis_index("c")
    s = lax.axis_index("s")
    wid = s * NC + c

    pltpu.sync_copy(zr_hbm.at[pl.ds(s * ROWS_PER_TILE, ROWS_PER_TILE)],
                    acc_sh.at[pl.ds(s * ROWS_PER_TILE, ROWS_PER_TILE)])

    @pl.when(s == 0)
    def _():
        pltpu.sync_copy(zr_hbm.at[pl.ds(0, ROWS_REM)],
                        acc_sh.at[pl.ds(NS * ROWS_PER_TILE, ROWS_REM)])
        pltpu.sync_copy(zc_hbm, den_sh)

    if weighted:
        pltpu.sync_copy(as_hbm, as_v)
        pltpu.sync_copy(ad_hbm, ad_v)
    else:
        for g in range(CHUNK // L):
            wa[pl.ds(g * L, L)] = jnp.ones((L,), jnp.float32)
            wb[pl.ds(g * L, L)] = jnp.ones((L,), jnp.float32)

    plsc.subcore_barrier()

    base_w = wid * N_CHUNKS

    def load_idx(ci, ibuf, idxd):
        base = (base_w + ci) * 2 * CHUNK
        pltpu.sync_copy(ed_hbm.at[pl.ds(base, 2 * CHUNK)], ibuf)
        for g in range(CHUNK // L):
            idxd[pl.ds(g * L, L)] = ibuf[pl.ds(CHUNK + g * L, L)]

    def start_gath(ibuf, rows_v, sem):
        pltpu.async_copy(rows_hbm.at[ibuf.at[pl.ds(0, CHUNK)]], rows_v, sem)

    def wait_gath(ibuf, rows_v, sem):
        pltpu.make_async_copy(rows_hbm.at[ibuf.at[pl.ds(0, CHUNK)]],
                              rows_v, sem).wait()

    def compute(rows_v, wv, ibuf, idxd):
        if not weighted:
            return

        def scale_group(g, carry2):
            sv = ibuf[pl.ds(g * L, L)]
            dv = idxd[pl.ds(g * L, L)]
            e = plsc.load_gather(as_v, [sv]) + plsc.load_gather(ad_v, [dv])
            e = jnp.where(e > 0, e, 0.2 * e)
            wgroup = jnp.exp(e)
            wv[pl.ds(g * L, L)] = wgroup
            for j16 in range(L):
                w = wgroup[j16]
                j = g * L + j16
                for r in range(D // L):
                    rows_v[j, pl.ds(r * L, L)] = rows_v[j, pl.ds(r * L, L)] * w
            return carry2

        lax.fori_loop(0, CHUNK // L, scale_group, 0)

    def start_scat(rows_v, wv, idxd, sem):
        pltpu.async_copy(rows_v, acc_sh.at[idxd], sem, add=True)
        pltpu.async_copy(wv, den_sh.at[idxd], sem, add=True)

    def wait_scat(rows_v, wv, idxd, sem):
        pltpu.make_async_copy(rows_v, acc_sh.at[idxd], sem).wait()
        pltpu.make_async_copy(wv, den_sh.at[idxd], sem).wait()

    load_idx(0, iba, ida)
    start_gath(iba, rows_a, sem_ga)
    load_idx(1, ibb, idb)
    start_gath(ibb, rows_b, sem_gb)

    def pair_body(k, carry):
        i0 = 2 * k
        wait_gath(iba, rows_a, sem_ga)
        compute(rows_a, wa, iba, ida)
        start_scat(rows_a, wa, ida, sem_sa)

        wait_gath(ibb, rows_b, sem_gb)
        compute(rows_b, wb, ibb, idb)
        start_scat(rows_b, wb, idb, sem_sb)

        @pl.when(i0 + 2 < N_CHUNKS)
        def _():
            wait_scat(rows_a, wa, ida, sem_sa)
            load_idx(i0 + 2, iba, ida)
            start_gath(iba, rows_a, sem_ga)

        @pl.when(i0 + 3 < N_CHUNKS)
        def _():
            wait_scat(rows_b, wb, idb, sem_sb)
            load_idx(i0 + 3, ibb, idb)
            start_gath(ibb, rows_b, sem_gb)

        return carry

    lax.fori_loop(0, N_CHUNKS // 2, pair_body, 0)
    if N_CHUNKS % 2:
        wait_gath(iba, rows_a, sem_ga)
        compute(rows_a, wa, iba, ida)
        start_scat(rows_a, wa, ida, sem_sa)
        wait_scat(rows_a, wa, ida, sem_sa)
    wait_scat(rows_b, wb, idb, sem_sb)
    if N_CHUNKS % 2 == 0:
        wait_scat(rows_a, wa, ida, sem_sa)

    plsc.subcore_barrier()

    pltpu.sync_copy(acc_sh.at[pl.ds(s * ROWS_PER_TILE, ROWS_PER_TILE)],
                    accp_hbm.at[c, pl.ds(s * ROWS_PER_TILE, ROWS_PER_TILE)])

    @pl.when(s == 0)
    def _():
        pltpu.sync_copy(acc_sh.at[pl.ds(NS * ROWS_PER_TILE, ROWS_REM)],
                        accp_hbm.at[c, pl.ds(NS * ROWS_PER_TILE, ROWS_REM)])
        pltpu.sync_copy(den_sh, denp_hbm.at[c])


def _edge_sweep(weighted, rows, ed, a_s=None, a_d=None):
    mesh = plsc.VectorSubcoreMesh(core_axis_name="c", subcore_axis_name="s",
                                  num_cores=NC, num_subcores=NS)
    scratch = [
        pltpu.VMEM((2 * CHUNK,), jnp.int32),
        pltpu.VMEM((CHUNK,), jnp.int32),
        pltpu.VMEM((2 * CHUNK,), jnp.int32),
        pltpu.VMEM((CHUNK,), jnp.int32),
        pltpu.VMEM((CHUNK, D), jnp.float32),
        pltpu.VMEM((CHUNK, D), jnp.float32),
        pltpu.VMEM((CHUNK,), jnp.float32),
        pltpu.VMEM((CHUNK,), jnp.float32),
        pltpu.VMEM((N_PAD if weighted else 8,), jnp.float32),
        pltpu.VMEM((N_PAD if weighted else 8,), jnp.float32),
        pltpu.VMEM_SHARED((N_PAD, D), jnp.float32),
        pltpu.VMEM_SHARED((N_PAD,), jnp.float32),
        pltpu.SemaphoreType.DMA,
        pltpu.SemaphoreType.DMA,
        pltpu.SemaphoreType.DMA,
        pltpu.SemaphoreType.DMA,
    ]
    fn = pl.kernel(
        functools.partial(_edge_sweep_body, weighted),
        out_type=(
            jax.ShapeDtypeStruct((NC, N_PAD, D), jnp.float32),
            jax.ShapeDtypeStruct((NC, N_PAD), jnp.float32),
        ),
        mesh=mesh,
        scratch_types=scratch,
        compiler_params=pltpu.CompilerParams(needs_layout_passes=False),
    )
    zr = jnp.zeros((N_PAD, D), jnp.float32)
    zc = jnp.zeros((N_PAD,), jnp.float32)
    if weighted:
        return fn(rows, ed, zr, zc,
                  jnp.pad(a_s, (0, N_PAD - N)), jnp.pad(a_d, (0, N_PAD - N)))
    return fn(rows, ed, zr, zc)


def _dense_mid_body(accp, cntp, x, wl, wr, gw, bl, ats, atd, hg_o, as_o, ad_o):
    agg = accp[0] + accp[1]
    cnt = cntp[0] + cntp[1]
    mean = agg / jnp.maximum(cnt, 1.0)
    dn = (((1,), (1,)), ((), ()))
    sage = (lax.dot_general(mean, wl[...], dn, preferred_element_type=jnp.float32)
            + bl[...]
            + lax.dot_general(x[...], wr[...], dn, preferred_element_type=jnp.float32))
    h = jnp.maximum(sage, 0.0)
    hg = lax.dot_general(h, gw[...], dn, preferred_element_type=jnp.float32)
    hg_o[...] = hg
    as_o[...] = jnp.sum(hg * ats[...], axis=1, keepdims=True)
    ad_o[...] = jnp.sum(hg * atd[...], axis=1, keepdims=True)


def _dense_final_body(nump, denp, hg, a_s, a_d, bias, out_o):
    num = nump[0] + nump[1]
    den = denp[0] + denp[1]
    es = a_s[...] + a_d[...]
    es = jnp.where(es > 0, es, 0.2 * es)
    ws = jnp.exp(es)
    num = num + ws * hg[...]
    den = den + ws
    o = num / (den + 1e-16) + bias[...]
    m = jnp.max(o, axis=1, keepdims=True)
    lse = jnp.log(jnp.sum(jnp.exp(o - m), axis=1, keepdims=True)) + m
    out_o[...] = o - lse


_BR = 1000
_GRID = N // _BR


def _dense_mid(accp, cntp, x, wl, wr, gw, bl, ats, atd):
    return pl.pallas_call(
        _dense_mid_body,
        grid=(_GRID,),
        in_specs=[
            pl.BlockSpec((NC, _BR, D), lambda i: (0, i, 0)),
            pl.BlockSpec((NC, _BR, 1), lambda i: (0, i, 0)),
            pl.BlockSpec((_BR, D), lambda i: (i, 0)),
            pl.BlockSpec((D, D), lambda i: (0, 0)),
            pl.BlockSpec((D, D), lambda i: (0, 0)),
            pl.BlockSpec((D, D), lambda i: (0, 0)),
            pl.BlockSpec((1, D), lambda i: (0, 0)),
            pl.BlockSpec((1, D), lambda i: (0, 0)),
            pl.BlockSpec((1, D), lambda i: (0, 0)),
        ],
        out_specs=[
            pl.BlockSpec((_BR, D), lambda i: (i, 0)),
            pl.BlockSpec((_BR, 1), lambda i: (i, 0)),
            pl.BlockSpec((_BR, 1), lambda i: (i, 0)),
        ],
        out_shape=[
            jax.ShapeDtypeStruct((N, D), jnp.float32),
            jax.ShapeDtypeStruct((N, 1), jnp.float32),
            jax.ShapeDtypeStruct((N, 1), jnp.float32),
        ],
    )(accp, cntp, x, wl, wr, gw, bl, ats, atd)


def _dense_final(nump, denp, hg, a_s, a_d, bias):
    return pl.pallas_call(
        _dense_final_body,
        grid=(_GRID,),
        in_specs=[
            pl.BlockSpec((NC, _BR, D), lambda i: (0, i, 0)),
            pl.BlockSpec((NC, _BR, 1), lambda i: (0, i, 0)),
            pl.BlockSpec((_BR, D), lambda i: (i, 0)),
            pl.BlockSpec((_BR, 1), lambda i: (i, 0)),
            pl.BlockSpec((_BR, 1), lambda i: (i, 0)),
            pl.BlockSpec((1, D), lambda i: (0, 0)),
        ],
        out_specs=pl.BlockSpec((_BR, D), lambda i: (i, 0)),
        out_shape=jax.ShapeDtypeStruct((N, D), jnp.float32),
    )(nump, denp, hg, a_s, a_d, bias)


def kernel(x, edge_index, sage_w_l, sage_b_l, sage_w_r, gat_w, gat_att_src,
           gat_att_dst, gat_bias):
    src = edge_index[0].astype(jnp.int32)
    dst = edge_index[1].astype(jnp.int32)

    ed = jnp.stack([src.reshape(NW, N_CHUNKS, CHUNK),
                    dst.reshape(NW, N_CHUNKS, CHUNK)], axis=2).reshape(-1)

    accp, cntp = _edge_sweep(False, x, ed)

    hg, a_s, a_d = _dense_mid(
        accp, cntp.reshape(NC, N_PAD, 1), x,
        sage_w_l, sage_w_r, gat_w,
        sage_b_l.reshape(1, D),
        gat_att_src.reshape(1, D), gat_att_dst.reshape(1, D))

    nump, denp = _edge_sweep(True, hg, ed,
                             a_s.reshape(N), a_d.reshape(N))

    return _dense_final(nump, denp.reshape(NC, N_PAD, 1), hg, a_s, a_d,
                        gat_bias.reshape(1, D))


# speedup vs baseline: 2.5258x; 1.1365x over previous
import functools

import jax
import jax.numpy as jnp
from jax import lax
from jax.experimental import pallas as pl
from jax.experimental.pallas import tpu as pltpu
from jax.experimental.pallas import tpu_sc as plsc

N = 10000
E = 320000
D = 128
NC, NS, L = 2, 16, 16
NW = NC * NS
E_PER_W = E // NW
CHUNK = 80
N_CHUNKS = E_PER_W // CHUNK
N_PAD = 10008
ROWS_PER_TILE = 624
ROWS_REM = N_PAD - NS * ROWS_PER_TILE


def _edge_sweep_body(weighted, *refs):
    if weighted:
        (rows_hbm, ed_hbm, zr_hbm, zc_hbm, as_hbm, ad_hbm,
         accp_hbm, denp_hbm, *rest) = refs
    else:
        (rows_hbm, ed_hbm, zr_hbm, zc_hbm,
         accp_hbm, denp_hbm, *rest) = refs
        as_hbm = ad_hbm = None
    (ib0, id0, rows0, w0, as0, ad0,
     ib1, id1, rows1, w1, as1, ad1,
     ib2, id2, rows2, w2, as2, ad2,
     acc_sh, den_sh,
     sg0, sg1, sg2, ss0, ss1, ss2) = rest
    set0 = (ib0, id0, rows0, w0, as0, ad0, sg0, ss0)
    set1 = (ib1, id1, rows1, w1, as1, ad1, sg1, ss1)
    set2 = (ib2, id2, rows2, w2, as2, ad2, sg2, ss2)

    c = lax.axis_index("c")
    s = lax.axis_index("s")
    wid = s * NC + c

    pltpu.sync_copy(zr_hbm.at[pl.ds(s * ROWS_PER_TILE, ROWS_PER_TILE)],
                    acc_sh.at[pl.ds(s * ROWS_PER_TILE, ROWS_PER_TILE)])

    @pl.when(s == 0)
    def _():
        pltpu.sync_copy(zr_hbm.at[pl.ds(0, ROWS_REM)],
                        acc_sh.at[pl.ds(NS * ROWS_PER_TILE, ROWS_REM)])
        pltpu.sync_copy(zc_hbm, den_sh)

    if not weighted:
        for g in range(CHUNK // L):
            w0[pl.ds(g * L, L)] = jnp.ones((L,), jnp.float32)
            w1[pl.ds(g * L, L)] = jnp.ones((L,), jnp.float32)
            w2[pl.ds(g * L, L)] = jnp.ones((L,), jnp.float32)

    plsc.subcore_barrier()

    base_w = wid * N_CHUNKS

    def load_idx(ci, S):
        ibuf, idxd = S[0], S[1]
        base = (base_w + ci) * 2 * CHUNK
        pltpu.sync_copy(ed_hbm.at[pl.ds(base, 2 * CHUNK)], ibuf)
        for g in range(CHUNK // L):
            idxd[pl.ds(g * L, L)] = ibuf[pl.ds(CHUNK + g * L, L)]

    def start_gath(S):
        ibuf, idxd, rows_v, _, asv, adv, sem = S[0], S[1], S[2], S[3], S[4], S[5], S[6]
        pltpu.async_copy(rows_hbm.at[ibuf.at[pl.ds(0, CHUNK)]], rows_v, sem)
        if weighted:
            pltpu.async_copy(as_hbm.at[ibuf.at[pl.ds(0, CHUNK)]], asv, sem)
            pltpu.async_copy(ad_hbm.at[idxd], adv, sem)

    def wait_gath(S):
        ibuf, idxd, rows_v, _, asv, adv, sem = S[0], S[1], S[2], S[3], S[4], S[5], S[6]
        pltpu.make_async_copy(rows_hbm.at[ibuf.at[pl.ds(0, CHUNK)]],
                              rows_v, sem).wait()
        if weighted:
            pltpu.make_async_copy(as_hbm.at[ibuf.at[pl.ds(0, CHUNK)]],
                                  asv, sem).wait()
            pltpu.make_async_copy(ad_hbm.at[idxd], adv, sem).wait()

    def compute(S):
        if not weighted:
            return
        rows_v, wv, asv, adv = S[2], S[3], S[4], S[5]

        def scale_group(g, carry2):
            e = asv[pl.ds(g * L, L)] + adv[pl.ds(g * L, L)]
            e = jnp.where(e > 0, e, 0.2 * e)
            wgroup = jnp.exp(e)
            wv[pl.ds(g * L, L)] = wgroup
            for j16 in range(L):
                w = wgroup[j16]
                j = g * L + j16
                for r in range(D // L):
                    rows_v[j, pl.ds(r * L, L)] = rows_v[j, pl.ds(r * L, L)] * w
            return carry2

        lax.fori_loop(0, CHUNK // L, scale_group, 0)

    def start_scat(S):
        idxd, rows_v, wv, sem = S[1], S[2], S[3], S[7]
        pltpu.async_copy(rows_v, acc_sh.at[idxd], sem, add=True)
        pltpu.async_copy(wv, den_sh.at[idxd], sem, add=True)

    def wait_scat(S):
        idxd, rows_v, wv, sem = S[1], S[2], S[3], S[7]
        pltpu.make_async_copy(rows_v, acc_sh.at[idxd], sem).wait()
        pltpu.make_async_copy(wv, den_sh.at[idxd], sem).wait()

    load_idx(0, set0)
    start_gath(set0)
    load_idx(1, set1)
    start_gath(set1)

    wait_gath(set0)
    compute(set0)
    start_scat(set0)
    load_idx(2, set2)
    start_gath(set2)

    def triple_body(k, carry):
        i = 3 * k + 1
        for t, (S, Sp) in enumerate(((set1, set0), (set2, set1), (set0, set2))):
            ii = i + t
            wait_gath(S)
            compute(S)
            start_scat(S)

            @pl.when(ii + 2 < N_CHUNKS)
            def _(S=S, Sp=Sp, ii=ii):
                wait_scat(Sp)
                load_idx(ii + 2, Sp)
                start_gath(Sp)

        return carry

    lax.fori_loop(0, (N_CHUNKS - 2) // 3, triple_body, 0)

    wait_gath(set1)
    compute(set1)
    start_scat(set1)
    wait_scat(set0)
    wait_scat(set1)
    wait_scat(set2)

    plsc.subcore_barrier()

    pltpu.sync_copy(acc_sh.at[pl.ds(s * ROWS_PER_TILE, ROWS_PER_TILE)],
                    accp_hbm.at[c, pl.ds(s * ROWS_PER_TILE, ROWS_PER_TILE)])

    @pl.when(s == 0)
    def _():
        pltpu.sync_copy(acc_sh.at[pl.ds(NS * ROWS_PER_TILE, ROWS_REM)],
                        accp_hbm.at[c, pl.ds(NS * ROWS_PER_TILE, ROWS_REM)])
        pltpu.sync_copy(den_sh, denp_hbm.at[c])


def _edge_sweep(weighted, rows, ed, a_s=None, a_d=None):
    mesh = plsc.VectorSubcoreMesh(core_axis_name="c", subcore_axis_name="s",
                                  num_cores=NC, num_subcores=NS)
    buf_set = [
        pltpu.VMEM((2 * CHUNK,), jnp.int32),
        pltpu.VMEM((CHUNK,), jnp.int32),
        pltpu.VMEM((CHUNK, D), jnp.float32),
        pltpu.VMEM((CHUNK,), jnp.float32),
        pltpu.VMEM((CHUNK,), jnp.float32),
        pltpu.VMEM((CHUNK,), jnp.float32),
    ]
    scratch = buf_set * 3 + [
        pltpu.VMEM_SHARED((N_PAD, D), jnp.float32),
        pltpu.VMEM_SHARED((N_PAD,), jnp.float32),
        pltpu.SemaphoreType.DMA,
        pltpu.SemaphoreType.DMA,
        pltpu.SemaphoreType.DMA,
        pltpu.SemaphoreType.DMA,
        pltpu.SemaphoreType.DMA,
        pltpu.SemaphoreType.DMA,
    ]
    fn = pl.kernel(
        functools.partial(_edge_sweep_body, weighted),
        out_type=(
            jax.ShapeDtypeStruct((NC, N_PAD, D), jnp.float32),
            jax.ShapeDtypeStruct((NC, N_PAD), jnp.float32),
        ),
        mesh=mesh,
        scratch_types=scratch,
        compiler_params=pltpu.CompilerParams(needs_layout_passes=False),
    )
    zr = jnp.zeros((N_PAD, D), jnp.float32)
    zc = jnp.zeros((N_PAD,), jnp.float32)
    if weighted:
        return fn(rows, ed, zr, zc,
                  jnp.pad(a_s, (0, N_PAD - N)), jnp.pad(a_d, (0, N_PAD - N)))
    return fn(rows, ed, zr, zc)


def _dense_mid_body(accp, cntp, x, wl, wr, gw, bl, ats, atd, hg_o, as_o, ad_o):
    agg = accp[0] + accp[1]
    cnt = cntp[0] + cntp[1]
    mean = agg / jnp.maximum(cnt, 1.0)
    dn = (((1,), (1,)), ((), ()))
    sage = (lax.dot_general(mean, wl[...], dn, preferred_element_type=jnp.float32)
            + bl[...]
            + lax.dot_general(x[...], wr[...], dn, preferred_element_type=jnp.float32))
    h = jnp.maximum(sage, 0.0)
    hg = lax.dot_general(h, gw[...], dn, preferred_element_type=jnp.float32)
    hg_o[...] = hg
    as_o[...] = jnp.sum(hg * ats[...], axis=1, keepdims=True)
    ad_o[...] = jnp.sum(hg * atd[...], axis=1, keepdims=True)


def _dense_final_body(nump, denp, hg, a_s, a_d, bias, out_o):
    num = nump[0] + nump[1]
    den = denp[0] + denp[1]
    es = a_s[...] + a_d[...]
    es = jnp.where(es > 0, es, 0.2 * es)
    ws = jnp.exp(es)
    num = num + ws * hg[...]
    den = den + ws
    o = num / (den + 1e-16) + bias[...]
    m = jnp.max(o, axis=1, keepdims=True)
    lse = jnp.log(jnp.sum(jnp.exp(o - m), axis=1, keepdims=True)) + m
    out_o[...] = o - lse


_BR = 1000
_GRID = N // _BR


def _dense_mid(accp, cntp, x, wl, wr, gw, bl, ats, atd):
    return pl.pallas_call(
        _dense_mid_body,
        grid=(_GRID,),
        in_specs=[
            pl.BlockSpec((NC, _BR, D), lambda i: (0, i, 0)),
            pl.BlockSpec((NC, _BR, 1), lambda i: (0, i, 0)),
            pl.BlockSpec((_BR, D), lambda i: (i, 0)),
            pl.BlockSpec((D, D), lambda i: (0, 0)),
            pl.BlockSpec((D, D), lambda i: (0, 0)),
            pl.BlockSpec((D, D), lambda i: (0, 0)),
            pl.BlockSpec((1, D), lambda i: (0, 0)),
            pl.BlockSpec((1, D), lambda i: (0, 0)),
            pl.BlockSpec((1, D), lambda i: (0, 0)),
        ],
        out_specs=[
            pl.BlockSpec((_BR, D), lambda i: (i, 0)),
            pl.BlockSpec((_BR, 1), lambda i: (i, 0)),
            pl.BlockSpec((_BR, 1), lambda i: (i, 0)),
        ],
        out_shape=[
            jax.ShapeDtypeStruct((N, D), jnp.float32),
            jax.ShapeDtypeStruct((N, 1), jnp.float32),
            jax.ShapeDtypeStruct((N, 1), jnp.float32),
        ],
    )(accp, cntp, x, wl, wr, gw, bl, ats, atd)


def _dense_final(nump, denp, hg, a_s, a_d, bias):
    return pl.pallas_call(
        _dense_final_body,
        grid=(_GRID,),
        in_specs=[
            pl.BlockSpec((NC, _BR, D), lambda i: (0, i, 0)),
            pl.BlockSpec((NC, _BR, 1), lambda i: (0, i, 0)),
            pl.BlockSpec((_BR, D), lambda i: (i, 0)),
            pl.BlockSpec((_BR, 1), lambda i: (i, 0)),
            pl.BlockSpec((_BR, 1), lambda i: (i, 0)),
            pl.BlockSpec((1, D), lambda i: (0, 0)),
        ],
        out_specs=pl.BlockSpec((_BR, D), lambda i: (i, 0)),
        out_shape=jax.ShapeDtypeStruct((N, D), jnp.float32),
    )(nump, denp, hg, a_s, a_d, bias)


def kernel(x, edge_index, sage_w_l, sage_b_l, sage_w_r, gat_w, gat_att_src,
           gat_att_dst, gat_bias):
    src = edge_index[0].astype(jnp.int32)
    dst = edge_index[1].astype(jnp.int32)

    ed = jnp.stack([src.reshape(NW, N_CHUNKS, CHUNK),
                    dst.reshape(NW, N_CHUNKS, CHUNK)], axis=2).reshape(-1)

    accp, cntp = _edge_sweep(False, x, ed)

    hg, a_s, a_d = _dense_mid(
        accp, cntp.reshape(NC, N_PAD, 1), x,
        sage_w_l, sage_w_r, gat_w,
        sage_b_l.reshape(1, D),
        gat_att_src.reshape(1, D), gat_att_dst.reshape(1, D))

    nump, denp = _edge_sweep(True, hg, ed,
                             a_s.reshape(N), a_d.reshape(N))

    return _dense_final(nump, denp.reshape(NC, N_PAD, 1), hg, a_s, a_d,
                        gat_bias.reshape(1, D))
